# Initial kernel scaffold; baseline (speedup 1.0000x reference)
import jax
import jax.numpy as jnp
from jax.experimental import pallas as pl


def kernel(x, edge_index, adj_vals, W, bias):
    raise NotImplementedError("write your pallas kernel here")


# speedup vs baseline: 4.0057x; 4.0057x over previous
import dataclasses
import functools

import jax
import jax.numpy as jnp
from jax import lax
from jax.experimental import pallas as pl
from jax.experimental.pallas import tpu as pltpu
from jax.experimental.pallas import tpu_sc as plsc

N = 10000
E = 320000
D = 128

NUM_CORES = 2
NUM_SUBCORES = 16
NUM_TILES = NUM_CORES * NUM_SUBCORES
EDGES_PER_TILE = E // NUM_TILES
CHUNK = 80
NCHUNKS = EDGES_PER_TILE // CHUNK
ROWS_PER_TILE = 624
ROWS_REM = N - NUM_SUBCORES * ROWS_PER_TILE
ZBLK = 208


def _mm_body(x_ref, w_ref, o_ref):
    o_ref[...] = jnp.dot(x_ref[...], w_ref[...],
                         preferred_element_type=jnp.float32)


def _support_matmul(x, W):
    return pl.pallas_call(
        _mm_body,
        grid=(10,),
        in_specs=[
            pl.BlockSpec((N // 10, D), lambda i: (i, 0)),
            pl.BlockSpec((D, D), lambda i: (0, 0)),
        ],
        out_specs=pl.BlockSpec((N // 10, D), lambda i: (i, 0)),
        out_shape=jax.ShapeDtypeStruct((N, D), jnp.float32),
    )(x, W)


def _sc_body(support_hbm, row_hbm, col_hbm, val_hbm, out_hbm,
             col_buf, row_buf, val_buf, rows_buf, zrow, acc):
    c = lax.axis_index("c")
    s = lax.axis_index("s")
    tile = c * NUM_SUBCORES + s

    @pl.loop(0, ZBLK)
    def _(i):
        for j in range(D // 16):
            zrow[i, pl.ds(j * 16, 16)] = jnp.zeros((16,), jnp.float32)

    r0 = s * ROWS_PER_TILE

    @pl.loop(0, ROWS_PER_TILE // ZBLK)
    def _(b):
        pltpu.sync_copy(zrow, acc.at[pl.ds(r0 + b * ZBLK, ZBLK)])

    @pl.when(s == 0)
    def _():
        pltpu.sync_copy(zrow.at[pl.ds(0, ROWS_REM)],
                        acc.at[pl.ds(N - ROWS_REM, ROWS_REM)])

    plsc.subcore_barrier()

    base0 = tile * EDGES_PER_TILE

    @pl.loop(0, NCHUNKS)
    def _(k):
        base = base0 + k * CHUNK
        pltpu.sync_copy(col_hbm.at[pl.ds(base, CHUNK)], col_buf)
        pltpu.sync_copy(row_hbm.at[pl.ds(base, CHUNK)], row_buf)
        pltpu.sync_copy(val_hbm.at[pl.ds(base, CHUNK)], val_buf)
        pltpu.sync_copy(support_hbm.at[col_buf], rows_buf)

        @pl.loop(0, CHUNK)
        def _(i):
            vv = plsc.load_gather(val_buf, [jnp.full((16,), i, jnp.int32)])
            for j in range(D // 16):
                sl = (i, pl.ds(j * 16, 16))
                rows_buf[sl] = rows_buf[sl] * vv

        pltpu.sync_copy(rows_buf, acc.at[row_buf], add=True)

    plsc.subcore_barrier()

    pltpu.sync_copy(acc.at[pl.ds(r0, ROWS_PER_TILE)],
                    out_hbm.at[c, pl.ds(r0, ROWS_PER_TILE)])

    @pl.when(s == 0)
    def _():
        pltpu.sync_copy(acc.at[pl.ds(N - ROWS_REM, ROWS_REM)],
                        out_hbm.at[c, pl.ds(N - ROWS_REM, ROWS_REM)])


def _sc_scatter(support, row, col, vals):
    mesh = plsc.VectorSubcoreMesh(core_axis_name="c", subcore_axis_name="s")
    cp = pltpu.CompilerParams()
    if "needs_layout_passes" in pltpu.CompilerParams.__dataclass_fields__:
        cp = dataclasses.replace(cp, needs_layout_passes=False)
    kern = pl.kernel(
        _sc_body,
        out_type=jax.ShapeDtypeStruct((NUM_CORES, N, D), jnp.float32),
        mesh=mesh,
        scratch_types=[
            pltpu.VMEM((CHUNK,), jnp.int32),
            pltpu.VMEM((CHUNK,), jnp.int32),
            pltpu.VMEM((CHUNK,), jnp.float32),
            pltpu.VMEM((CHUNK, D), jnp.float32),
            pltpu.VMEM((ZBLK, D), jnp.float32),
            pltpu.VMEM_SHARED((N, D), jnp.float32),
        ],
        compiler_params=cp,
    )
    return kern(support, row, col, vals)


def _combine_body(p_ref, b_ref, o_ref):
    o_ref[...] = p_ref[0] + p_ref[1] + b_ref[...]


def _combine(partials, bias2d):
    return pl.pallas_call(
        _combine_body,
        grid=(10,),
        in_specs=[
            pl.BlockSpec((NUM_CORES, N // 10, D), lambda i: (0, i, 0)),
            pl.BlockSpec((1, D), lambda i: (0, 0)),
        ],
        out_specs=pl.BlockSpec((N // 10, D), lambda i: (i, 0)),
        out_shape=jax.ShapeDtypeStruct((N, D), jnp.float32),
    )(partials, bias2d)


def kernel(x, edge_index, adj_vals, W, bias):
    support = _support_matmul(x, W)
    row = edge_index[0]
    col = edge_index[1]
    partials = _sc_scatter(support, row, col, adj_vals)
    return _combine(partials, bias.reshape(1, D))


# speedup vs baseline: 5.0389x; 1.2579x over previous
import dataclasses
import functools

import jax
import jax.numpy as jnp
from jax import lax
from jax.experimental import pallas as pl
from jax.experimental.pallas import tpu as pltpu
from jax.experimental.pallas import tpu_sc as plsc

N = 10000
E = 320000
D = 128
DH = D // 2

NUM_CORES = 2
NUM_SUBCORES = 16
EDGES_PER_SUBCORE = E // NUM_SUBCORES
CHUNK = 80
NCHUNKS = EDGES_PER_SUBCORE // CHUNK
NBLOCKS = 10
BCH = NCHUNKS // NBLOCKS
NBUF = 5
LOOK = 3
ROWS_PER_TILE = 624
ROWS_REM = N - NUM_SUBCORES * ROWS_PER_TILE
ZBLK = 104


def _mm_body(x_ref, w_ref, o_ref):
    o_ref[0] = jnp.dot(x_ref[...], w_ref[0],
                       preferred_element_type=jnp.float32)


def _support_matmul(x, W2):
    return pl.pallas_call(
        _mm_body,
        grid=(NUM_CORES, 10),
        in_specs=[
            pl.BlockSpec((N // 10, D), lambda c, i: (i, 0)),
            pl.BlockSpec((1, D, DH), lambda c, i: (c, 0, 0)),
        ],
        out_specs=pl.BlockSpec((1, N // 10, DH), lambda c, i: (c, i, 0)),
        out_shape=jax.ShapeDtypeStruct((NUM_CORES, N, DH), jnp.float32),
    )(x, W2)


def _sc_body(sup_hbm, row_hbm, col_hbm, val_hbm, out_hbm,
             col_blk, row_blk, val_blk, zrow, bufs, gsems, ssems, isems, acc):
    c = lax.axis_index("c")
    s = lax.axis_index("s")

    my_sup = sup_hbm.at[c]
    my_out = out_hbm.at[c]
    r0 = s * ROWS_PER_TILE

    pltpu.sync_copy(my_sup.at[pl.ds(r0, ROWS_PER_TILE)],
                    acc.at[0].at[pl.ds(r0, ROWS_PER_TILE)])

    @pl.loop(0, ZBLK)
    def _(i):
        for j in range(DH // 16):
            zrow[i, pl.ds(j * 16, 16)] = jnp.zeros((16,), jnp.float32)

    @pl.loop(0, ROWS_PER_TILE // ZBLK)
    def _(b):
        pltpu.sync_copy(zrow, acc.at[1].at[pl.ds(r0 + b * ZBLK, ZBLK)])

    @pl.when(s == 0)
    def _():
        pltpu.sync_copy(my_sup.at[pl.ds(N - ROWS_REM, ROWS_REM)],
                        acc.at[0].at[pl.ds(N - ROWS_REM, ROWS_REM)])
        pltpu.sync_copy(zrow.at[pl.ds(0, ROWS_REM)],
                        acc.at[1].at[pl.ds(N - ROWS_REM, ROWS_REM)])

    pltpu.sync_copy(col_hbm.at[s, 0], col_blk.at[0])
    pltpu.sync_copy(row_hbm.at[s, 0], row_blk.at[0])
    pltpu.sync_copy(val_hbm.at[s, 0], val_blk.at[0])

    plsc.subcore_barrier()

    sup_s = acc.at[0]
    acc_s = acc.at[1]
    full16 = lambda v: jnp.full((16,), v, jnp.int32)

    def compute_sync(par, k, b):
        @pl.loop(0, CHUNK)
        def _(i):
            vv = plsc.load_gather(val_blk, [full16(par), full16(k),
                                            full16(i)])
            for j in range(DH // 16):
                bufs[b, i, pl.ds(j * 16, 16)] = (
                    bufs[b, i, pl.ds(j * 16, 16)] * vv)

    @pl.loop(0, NBLOCKS)
    def _(bb):
        @pl.when(bb > 0)
        def _():
            pltpu.sync_copy(col_hbm.at[s, bb], col_blk.at[0])
            pltpu.sync_copy(row_hbm.at[s, bb], row_blk.at[0])
            pltpu.sync_copy(val_hbm.at[s, bb], val_blk.at[0])

        @pl.loop(0, BCH)
        def _(k):
            pltpu.sync_copy(sup_s.at[col_blk.at[0, k]], bufs.at[0])
            compute_sync(0, k, 0)
            pltpu.sync_copy(bufs.at[0], acc_s.at[row_blk.at[0, k]],
                            add=True)

    plsc.subcore_barrier()

    pltpu.sync_copy(acc_s.at[pl.ds(r0, ROWS_PER_TILE)],
                    my_out.at[pl.ds(r0, ROWS_PER_TILE)])

    @pl.when(s == 0)
    def _():
        pltpu.sync_copy(acc_s.at[pl.ds(N - ROWS_REM, ROWS_REM)],
                        my_out.at[pl.ds(N - ROWS_REM, ROWS_REM)])
    return

    def issue_gather(par, k, b):
        pltpu.async_copy(sup_s.at[col_blk.at[par, k]], bufs.at[b],
                         gsems.at[b])

    def wait_gather(b):
        pltpu.make_async_copy(my_sup.at[pl.ds(0, CHUNK)], bufs.at[b],
                              gsems.at[b]).wait()

    def issue_scatter(par, k, b):
        pltpu.async_copy(bufs.at[b], acc_s.at[row_blk.at[par, k]],
                         ssems.at[b], add=True)

    def wait_scatter(b):
        pltpu.make_async_copy(my_sup.at[pl.ds(0, CHUNK)],
                              acc_s.at[pl.ds(0, CHUNK)], ssems.at[b]).wait()

    def issue_idx(bb1, par1):
        pltpu.async_copy(col_hbm.at[s, bb1], col_blk.at[par1],
                         isems.at[par1])
        pltpu.async_copy(row_hbm.at[s, bb1], row_blk.at[par1],
                         isems.at[par1])
        pltpu.async_copy(val_hbm.at[s, bb1], val_blk.at[par1],
                         isems.at[par1])

    def wait_idx(par1):
        pltpu.make_async_copy(col_hbm.at[s, 0], col_blk.at[par1],
                              isems.at[par1]).wait()
        pltpu.make_async_copy(row_hbm.at[s, 0], row_blk.at[par1],
                              isems.at[par1]).wait()
        pltpu.make_async_copy(val_hbm.at[s, 0], val_blk.at[par1],
                              isems.at[par1]).wait()

    def compute(par, k, b):
        @pl.loop(0, CHUNK)
        def _(i):
            vv = plsc.load_gather(val_blk, [full16(par), full16(k),
                                            full16(i)])
            for j in range(DH // 16):
                bufs[b, i, pl.ds(j * 16, 16)] = (
                    bufs[b, i, pl.ds(j * 16, 16)] * vv)

    @pl.loop(0, NBLOCKS // 2)
    def _(bp):
        for par in range(2):
            bb = 2 * bp + par
            if par == 0:
                @pl.when(bp > 0)
                def _():
                    wait_idx(0)
            else:
                wait_idx(1)

            for b in range(LOOK):
                if par == 0:
                    @pl.when(bp > 0)
                    def _():
                        wait_scatter(b)
                else:
                    wait_scatter(b)
                issue_gather(par, b, b)

            @pl.loop(0, BCH // NBUF)
            def _(g5):
                for b5 in range(NBUF):
                    k = g5 * NBUF + b5
                    if b5 == 0:
                        @pl.when((g5 == 1) & (bb + 1 < NBLOCKS))
                        def _():
                            issue_idx(bb + 1, 1 - par)
                    wait_gather(b5)
                    compute(par, k, b5)
                    issue_scatter(par, k, b5)

                    tgt = (b5 + LOOK) % NBUF
                    if b5 < NBUF - LOOK:
                        if par == 0:
                            @pl.when((bp > 0) | (g5 > 0))
                            def _():
                                wait_scatter(tgt)

                            issue_gather(par, k + LOOK, tgt)
                        else:
                            wait_scatter(tgt)
                            issue_gather(par, k + LOOK, tgt)
                    else:
                        @pl.when(g5 < BCH // NBUF - 1)
                        def _():
                            wait_scatter(tgt)
                            issue_gather(par, k + LOOK, tgt)

    for b in range(NBUF):
        wait_scatter(b)

    plsc.subcore_barrier()

    pltpu.sync_copy(acc_s.at[pl.ds(r0, ROWS_PER_TILE)],
                    my_out.at[pl.ds(r0, ROWS_PER_TILE)])

    @pl.when(s == 0)
    def _():
        pltpu.sync_copy(acc_s.at[pl.ds(N - ROWS_REM, ROWS_REM)],
                        my_out.at[pl.ds(N - ROWS_REM, ROWS_REM)])


def _sc_scatter(sup_halves, row3, col3, val3):
    mesh = plsc.VectorSubcoreMesh(core_axis_name="c", subcore_axis_name="s")
    cp = pltpu.CompilerParams(use_tc_tiling_on_sc=False)
    if "needs_layout_passes" in pltpu.CompilerParams.__dataclass_fields__:
        cp = dataclasses.replace(cp, needs_layout_passes=False)
    kern = pl.kernel(
        _sc_body,
        out_type=jax.ShapeDtypeStruct((NUM_CORES, N, DH), jnp.float32),
        mesh=mesh,
        scratch_types=[
            pltpu.VMEM((2, BCH, CHUNK), jnp.int32),
            pltpu.VMEM((2, BCH, CHUNK), jnp.int32),
            pltpu.VMEM((2, BCH, CHUNK), jnp.float32),
            pltpu.VMEM((ZBLK, DH), jnp.float32),
            pltpu.VMEM((NBUF, CHUNK, DH), jnp.float32),
            pltpu.SemaphoreType.DMA((NBUF,)),
            pltpu.SemaphoreType.DMA((NBUF,)),
            pltpu.SemaphoreType.DMA((2,)),
            pltpu.VMEM_SHARED((2, N, DH), jnp.float32),
        ],
        compiler_params=cp,
    )
    return kern(sup_halves, row3, col3, val3)


def _combine_body(p_ref, b_ref, o_ref):
    o_ref[...] = jnp.concatenate([p_ref[0], p_ref[1]], axis=1) + b_ref[...]


def _combine(partials, bias2d):
    return pl.pallas_call(
        _combine_body,
        grid=(10,),
        in_specs=[
            pl.BlockSpec((NUM_CORES, N // 10, DH), lambda i: (0, i, 0)),
            pl.BlockSpec((1, D), lambda i: (0, 0)),
        ],
        out_specs=pl.BlockSpec((N // 10, D), lambda i: (i, 0)),
        out_shape=jax.ShapeDtypeStruct((N, D), jnp.float32),
    )(partials, bias2d)


def kernel(x, edge_index, adj_vals, W, bias):
    W2 = W.reshape(D, NUM_CORES, DH).transpose(1, 0, 2)
    support = _support_matmul(x, W2)
    row3 = edge_index[0].reshape(NUM_SUBCORES, NBLOCKS, BCH, CHUNK)
    col3 = edge_index[1].reshape(NUM_SUBCORES, NBLOCKS, BCH, CHUNK)
    val3 = adj_vals.reshape(NUM_SUBCORES, NBLOCKS, BCH, CHUNK)
    partials = _sc_scatter(support, row3, col3, val3)
    return _combine(partials, bias.reshape(1, D))


# speedup vs baseline: 8.3750x; 1.6621x over previous
import dataclasses
import functools

import jax
import jax.numpy as jnp
from jax import lax
from jax.experimental import pallas as pl
from jax.experimental.pallas import tpu as pltpu
from jax.experimental.pallas import tpu_sc as plsc

N = 10000
E = 320000
D = 128
DH = D // 2

NUM_CORES = 2
NUM_SUBCORES = 16
EDGES_PER_SUBCORE = E // NUM_SUBCORES
CHUNK = 80
NCHUNKS = EDGES_PER_SUBCORE // CHUNK
NBLOCKS = 10
BCH = NCHUNKS // NBLOCKS
NBUF = 5
LOOK = 3
ROWS_PER_TILE = 624
ROWS_REM = N - NUM_SUBCORES * ROWS_PER_TILE
ZBLK = 104


def _mm_body(x_ref, w_ref, o_ref):
    o_ref[0] = jnp.dot(x_ref[...], w_ref[0],
                       preferred_element_type=jnp.float32)


def _support_matmul(x, W2):
    return pl.pallas_call(
        _mm_body,
        grid=(NUM_CORES, 10),
        in_specs=[
            pl.BlockSpec((N // 10, D), lambda c, i: (i, 0)),
            pl.BlockSpec((1, D, DH), lambda c, i: (c, 0, 0)),
        ],
        out_specs=pl.BlockSpec((1, N // 10, DH), lambda c, i: (c, i, 0)),
        out_shape=jax.ShapeDtypeStruct((NUM_CORES, N, DH), jnp.float32),
    )(x, W2)


def _sc_body(sup_hbm, row_hbm, col_hbm, val_hbm, out_hbm,
             col_blk, row_blk, val_blk, zrow, bufs, gsems, ssems, isems, acc):
    c = lax.axis_index("c")
    s = lax.axis_index("s")

    my_sup = sup_hbm.at[c]
    my_out = out_hbm.at[c]
    r0 = s * ROWS_PER_TILE

    pltpu.sync_copy(my_sup.at[pl.ds(r0, ROWS_PER_TILE)],
                    acc.at[0].at[pl.ds(r0, ROWS_PER_TILE)])

    @pl.loop(0, ZBLK)
    def _(i):
        for j in range(DH // 16):
            zrow[i, pl.ds(j * 16, 16)] = jnp.zeros((16,), jnp.float32)

    @pl.loop(0, ROWS_PER_TILE // ZBLK)
    def _(b):
        pltpu.sync_copy(zrow, acc.at[1].at[pl.ds(r0 + b * ZBLK, ZBLK)])

    @pl.when(s == 0)
    def _():
        pltpu.sync_copy(my_sup.at[pl.ds(N - ROWS_REM, ROWS_REM)],
                        acc.at[0].at[pl.ds(N - ROWS_REM, ROWS_REM)])
        pltpu.sync_copy(zrow.at[pl.ds(0, ROWS_REM)],
                        acc.at[1].at[pl.ds(N - ROWS_REM, ROWS_REM)])

    pltpu.sync_copy(col_hbm.at[s, 0], col_blk.at[0])
    pltpu.sync_copy(row_hbm.at[s, 0], row_blk.at[0])
    pltpu.sync_copy(val_hbm.at[s, 0], val_blk.at[0])

    plsc.subcore_barrier()

    sup_s = acc.at[0]
    acc_s = acc.at[1]
    full16 = lambda v: jnp.full((16,), v, jnp.int32)

    def issue_gather(par, k, b):
        pltpu.async_copy(sup_s.at[col_blk.at[par, k]], bufs.at[b],
                         gsems.at[b])

    def wait_gather(par, k, b):
        pltpu.make_async_copy(sup_s.at[col_blk.at[par, k]], bufs.at[b],
                              gsems.at[b]).wait()

    def issue_scatter(par, k, b):
        pltpu.async_copy(bufs.at[b], acc_s.at[row_blk.at[par, k]],
                         ssems.at[b], add=True)

    def wait_scatter(par, k, b):
        pltpu.make_async_copy(bufs.at[b], acc_s.at[row_blk.at[par, k]],
                              ssems.at[b]).wait()

    def issue_idx(bb1, par1):
        pltpu.async_copy(col_hbm.at[s, bb1], col_blk.at[par1],
                         isems.at[par1])
        pltpu.async_copy(row_hbm.at[s, bb1], row_blk.at[par1],
                         isems.at[par1])
        pltpu.async_copy(val_hbm.at[s, bb1], val_blk.at[par1],
                         isems.at[par1])

    def wait_idx(par1):
        pltpu.make_async_copy(col_hbm.at[s, 0], col_blk.at[par1],
                              isems.at[par1]).wait()
        pltpu.make_async_copy(row_hbm.at[s, 0], row_blk.at[par1],
                              isems.at[par1]).wait()
        pltpu.make_async_copy(val_hbm.at[s, 0], val_blk.at[par1],
                              isems.at[par1]).wait()

    def compute(par, k, b):
        @pl.loop(0, CHUNK)
        def _(i):
            vv = plsc.load_gather(val_blk, [full16(par), full16(k),
                                            full16(i)])
            for j in range(DH // 16):
                bufs[b, i, pl.ds(j * 16, 16)] = (
                    bufs[b, i, pl.ds(j * 16, 16)] * vv)

    @pl.loop(0, NBLOCKS // 2)
    def _(bp):
        for par in range(2):
            bb = 2 * bp + par
            if par == 0:
                @pl.when(bp > 0)
                def _():
                    wait_idx(0)
            else:
                wait_idx(1)

            for b in range(LOOK):
                if par == 0:
                    @pl.when(bp > 0)
                    def _():
                        wait_scatter(par, 0, b)
                else:
                    wait_scatter(par, 0, b)
                issue_gather(par, b, b)

            @pl.loop(0, BCH // NBUF)
            def _(g5):
                for b5 in range(NBUF):
                    k = g5 * NBUF + b5
                    if b5 == 0:
                        @pl.when((g5 == 1) & (bb + 1 < NBLOCKS))
                        def _():
                            issue_idx(bb + 1, 1 - par)
                    wait_gather(par, k, b5)
                    compute(par, k, b5)
                    issue_scatter(par, k, b5)

                    tgt = (b5 + LOOK) % NBUF
                    if b5 < NBUF - LOOK:
                        if par == 0:
                            @pl.when((bp > 0) | (g5 > 0))
                            def _():
                                wait_scatter(par, k, tgt)

                            issue_gather(par, k + LOOK, tgt)
                        else:
                            wait_scatter(par, k, tgt)
                            issue_gather(par, k + LOOK, tgt)
                    else:
                        @pl.when(g5 < BCH // NBUF - 1)
                        def _():
                            wait_scatter(par, k, tgt)
                            issue_gather(par, k + LOOK, tgt)

    for b in range(NBUF):
        wait_scatter(1, 0, b)

    plsc.subcore_barrier()

    pltpu.sync_copy(acc_s.at[pl.ds(r0, ROWS_PER_TILE)],
                    my_out.at[pl.ds(r0, ROWS_PER_TILE)])

    @pl.when(s == 0)
    def _():
        pltpu.sync_copy(acc_s.at[pl.ds(N - ROWS_REM, ROWS_REM)],
                        my_out.at[pl.ds(N - ROWS_REM, ROWS_REM)])


def _sc_scatter(sup_halves, row3, col3, val3):
    mesh = plsc.VectorSubcoreMesh(core_axis_name="c", subcore_axis_name="s")
    cp = pltpu.CompilerParams(use_tc_tiling_on_sc=False)
    if "needs_layout_passes" in pltpu.CompilerParams.__dataclass_fields__:
        cp = dataclasses.replace(cp, needs_layout_passes=False)
    kern = pl.kernel(
        _sc_body,
        out_type=jax.ShapeDtypeStruct((NUM_CORES, N, DH), jnp.float32),
        mesh=mesh,
        scratch_types=[
            pltpu.VMEM((2, BCH, CHUNK), jnp.int32),
            pltpu.VMEM((2, BCH, CHUNK), jnp.int32),
            pltpu.VMEM((2, BCH, CHUNK), jnp.float32),
            pltpu.VMEM((ZBLK, DH), jnp.float32),
            pltpu.VMEM((NBUF, CHUNK, DH), jnp.float32),
            pltpu.SemaphoreType.DMA((NBUF,)),
            pltpu.SemaphoreType.DMA((NBUF,)),
            pltpu.SemaphoreType.DMA((2,)),
            pltpu.VMEM_SHARED((2, N, DH), jnp.float32),
        ],
        compiler_params=cp,
    )
    return kern(sup_halves, row3, col3, val3)


def _combine_body(p_ref, b_ref, o_ref):
    o_ref[...] = jnp.concatenate([p_ref[0], p_ref[1]], axis=1) + b_ref[...]


def _combine(partials, bias2d):
    return pl.pallas_call(
        _combine_body,
        grid=(10,),
        in_specs=[
            pl.BlockSpec((NUM_CORES, N // 10, DH), lambda i: (0, i, 0)),
            pl.BlockSpec((1, D), lambda i: (0, 0)),
        ],
        out_specs=pl.BlockSpec((N // 10, D), lambda i: (i, 0)),
        out_shape=jax.ShapeDtypeStruct((N, D), jnp.float32),
    )(partials, bias2d)


def kernel(x, edge_index, adj_vals, W, bias):
    W2 = W.reshape(D, NUM_CORES, DH).transpose(1, 0, 2)
    support = _support_matmul(x, W2)
    row3 = edge_index[0].reshape(NUM_SUBCORES, NBLOCKS, BCH, CHUNK)
    col3 = edge_index[1].reshape(NUM_SUBCORES, NBLOCKS, BCH, CHUNK)
    val3 = adj_vals.reshape(NUM_SUBCORES, NBLOCKS, BCH, CHUNK)
    partials = _sc_scatter(support, row3, col3, val3)
    return _combine(partials, bias.reshape(1, D))


# speedup vs baseline: 8.5597x; 1.0221x over previous
import dataclasses
import functools

import jax
import jax.numpy as jnp
from jax import lax
from jax.experimental import pallas as pl
from jax.experimental.pallas import tpu as pltpu
from jax.experimental.pallas import tpu_sc as plsc

N = 10000
E = 320000
D = 128
DH = D // 2

NUM_CORES = 2
NUM_SUBCORES = 16
EDGES_PER_SUBCORE = E // NUM_SUBCORES
CHUNK = 80
NCHUNKS = EDGES_PER_SUBCORE // CHUNK
NBLOCKS = 10
BCH = NCHUNKS // NBLOCKS
BLKE = BCH * CHUNK
NBUF = 5
LOOK = 3
ROWS_PER_TILE = 624
ROWS_REM = N - NUM_SUBCORES * ROWS_PER_TILE
ZBLK = 104


def _mm_body(x_ref, w_ref, o_ref):
    o_ref[0] = jnp.dot(x_ref[...], w_ref[0],
                       preferred_element_type=jnp.float32,
                       precision=lax.Precision.DEFAULT)
    o_ref[1] = jnp.dot(x_ref[...], w_ref[1],
                       preferred_element_type=jnp.float32,
                       precision=lax.Precision.DEFAULT)


def _support_matmul(x, W2):
    return pl.pallas_call(
        _mm_body,
        grid=(10,),
        in_specs=[
            pl.BlockSpec((N // 10, D), lambda i: (i, 0)),
            pl.BlockSpec((NUM_CORES, D, DH), lambda i: (0, 0, 0)),
        ],
        out_specs=pl.BlockSpec((NUM_CORES, N // 10, DH), lambda i: (0, i, 0)),
        out_shape=jax.ShapeDtypeStruct((NUM_CORES, N, DH), jnp.float32),
    )(x, W2)


def _sc_body(sup_hbm, row_hbm, col_hbm, val_hbm, bias_hbm, out_hbm,
             col_blk, row_blk, val_blk, zrow, bufs, gsems, ssems, isems, acc):
    c = lax.axis_index("c")
    s = lax.axis_index("s")

    my_sup = sup_hbm.at[c]
    my_out = out_hbm.at[c]
    r0 = s * ROWS_PER_TILE

    pltpu.sync_copy(my_sup.at[pl.ds(r0, ROWS_PER_TILE)],
                    acc.at[0].at[pl.ds(r0, ROWS_PER_TILE)])

    pltpu.sync_copy(bias_hbm.at[c], zrow.at[0])
    bvec = [zrow[0, pl.ds(j * 16, 16)] for j in range(DH // 16)]

    @pl.loop(1, ZBLK)
    def _(i):
        for j in range(DH // 16):
            zrow[i, pl.ds(j * 16, 16)] = bvec[j]

    @pl.loop(0, ROWS_PER_TILE // ZBLK)
    def _(b):
        pltpu.sync_copy(zrow, acc.at[1].at[pl.ds(r0 + b * ZBLK, ZBLK)])

    @pl.when(s == 0)
    def _():
        pltpu.sync_copy(my_sup.at[pl.ds(N - ROWS_REM, ROWS_REM)],
                        acc.at[0].at[pl.ds(N - ROWS_REM, ROWS_REM)])
        pltpu.sync_copy(zrow.at[pl.ds(0, ROWS_REM)],
                        acc.at[1].at[pl.ds(N - ROWS_REM, ROWS_REM)])

    base0 = s * EDGES_PER_SUBCORE
    pltpu.sync_copy(col_hbm.at[s, 0], col_blk.at[0])
    pltpu.sync_copy(row_hbm.at[s, 0], row_blk.at[0])
    pltpu.sync_copy(val_hbm.at[s, 0], val_blk.at[0])

    plsc.subcore_barrier()

    sup_s = acc.at[0]
    acc_s = acc.at[1]
    full16 = lambda v: jnp.full((16,), v, jnp.int32)

    def issue_gather(par, k, b):
        pltpu.async_copy(
            sup_s.at[col_blk.at[par, k]],
            bufs.at[b], gsems.at[b])

    def wait_gather(par, k, b):
        pltpu.make_async_copy(
            sup_s.at[col_blk.at[par, k]],
            bufs.at[b], gsems.at[b]).wait()

    def issue_scatter(par, k, b):
        pltpu.async_copy(
            bufs.at[b],
            acc_s.at[row_blk.at[par, k]],
            ssems.at[b], add=True)

    def wait_scatter(par, k, b):
        pltpu.make_async_copy(
            bufs.at[b],
            acc_s.at[row_blk.at[par, k]],
            ssems.at[b]).wait()

    def issue_idx(bb1, par1):
        pltpu.async_copy(col_hbm.at[s, bb1], col_blk.at[par1],
                         isems.at[par1])
        pltpu.async_copy(row_hbm.at[s, bb1], row_blk.at[par1],
                         isems.at[par1])
        pltpu.async_copy(val_hbm.at[s, bb1], val_blk.at[par1],
                         isems.at[par1])

    def wait_idx(par1):
        pltpu.make_async_copy(col_hbm.at[s, 0], col_blk.at[par1],
                              isems.at[par1]).wait()
        pltpu.make_async_copy(row_hbm.at[s, 0], row_blk.at[par1],
                              isems.at[par1]).wait()
        pltpu.make_async_copy(val_hbm.at[s, 0], val_blk.at[par1],
                              isems.at[par1]).wait()

    def compute(par, k, b):
        @pl.loop(0, CHUNK)
        def _(i):
            vv = plsc.load_gather(val_blk, [full16(par), full16(k),
                                            full16(i)])
            for j in range(DH // 16):
                bufs[b, i, pl.ds(j * 16, 16)] = (
                    bufs[b, i, pl.ds(j * 16, 16)] * vv)

    @pl.loop(0, NBLOCKS // 2)
    def _(bp):
        for par in range(2):
            bb = 2 * bp + par
            if par == 0:
                @pl.when(bp > 0)
                def _():
                    wait_idx(0)
            else:
                wait_idx(1)

            for b in range(LOOK):
                if par == 0:
                    @pl.when(bp > 0)
                    def _():
                        wait_scatter(par, 0, b)
                else:
                    wait_scatter(par, 0, b)
                issue_gather(par, b, b)

            @pl.loop(0, BCH // NBUF)
            def _(g5):
                for b5 in range(NBUF):
                    k = g5 * NBUF + b5
                    if b5 == 0:
                        @pl.when((g5 == 1) & (bb + 1 < NBLOCKS))
                        def _():
                            issue_idx(bb + 1, 1 - par)
                    wait_gather(par, k, b5)
                    compute(par, k, b5)
                    issue_scatter(par, k, b5)

                    tgt = (b5 + LOOK) % NBUF
                    if b5 < NBUF - LOOK:
                        if par == 0:
                            @pl.when((bp > 0) | (g5 > 0))
                            def _():
                                wait_scatter(par, k, tgt)

                            issue_gather(par, k + LOOK, tgt)
                        else:
                            wait_scatter(par, k, tgt)
                            issue_gather(par, k + LOOK, tgt)
                    else:
                        @pl.when(g5 < BCH // NBUF - 1)
                        def _():
                            wait_scatter(par, k, tgt)
                            issue_gather(par, k + LOOK, tgt)

    for b in range(NBUF):
        wait_scatter(1, 0, b)

    plsc.subcore_barrier()

    pltpu.sync_copy(acc_s.at[pl.ds(r0, ROWS_PER_TILE)],
                    my_out.at[pl.ds(r0, ROWS_PER_TILE)])

    @pl.when(s == 0)
    def _():
        pltpu.sync_copy(acc_s.at[pl.ds(N - ROWS_REM, ROWS_REM)],
                        my_out.at[pl.ds(N - ROWS_REM, ROWS_REM)])


def _sc_scatter(sup_halves, row3, col3, val3, bias2):
    mesh = plsc.VectorSubcoreMesh(core_axis_name="c", subcore_axis_name="s")
    cp = pltpu.CompilerParams(use_tc_tiling_on_sc=False)
    if "needs_layout_passes" in pltpu.CompilerParams.__dataclass_fields__:
        cp = dataclasses.replace(cp, needs_layout_passes=False)
    kern = pl.kernel(
        _sc_body,
        out_type=jax.ShapeDtypeStruct((NUM_CORES, N, DH), jnp.float32),
        mesh=mesh,
        scratch_types=[
            pltpu.VMEM((2, BCH, CHUNK), jnp.int32),
            pltpu.VMEM((2, BCH, CHUNK), jnp.int32),
            pltpu.VMEM((2, BCH, CHUNK), jnp.float32),
            pltpu.VMEM((ZBLK, DH), jnp.float32),
            pltpu.VMEM((NBUF, CHUNK, DH), jnp.float32),
            pltpu.SemaphoreType.DMA((NBUF,)),
            pltpu.SemaphoreType.DMA((NBUF,)),
            pltpu.SemaphoreType.DMA((2,)),
            pltpu.VMEM_SHARED((2, N, DH), jnp.float32),
        ],
        compiler_params=cp,
    )
    return kern(sup_halves, row3, col3, val3, bias2)


def kernel(x, edge_index, adj_vals, W, bias):
    W2 = W.reshape(D, NUM_CORES, DH).transpose(1, 0, 2)
    support = _support_matmul(x, W2)
    row3 = edge_index[0].reshape(NUM_SUBCORES, NBLOCKS, BCH, CHUNK)
    col3 = edge_index[1].reshape(NUM_SUBCORES, NBLOCKS, BCH, CHUNK)
    val3 = adj_vals.reshape(NUM_SUBCORES, NBLOCKS, BCH, CHUNK)
    bias2 = bias.reshape(NUM_CORES, DH)
    partials = _sc_scatter(support, row3, col3, val3, bias2)
    return partials.transpose(1, 0, 2).reshape(N, D)
